# baseline (speedup 1.0000x reference)
import functools

import jax
import jax.numpy as jnp
from jax import lax
from jax.experimental import pallas as pl
from jax.experimental.pallas import tpu as pltpu
from jax.experimental.pallas import tpu_sc as plsc

_SP_CHUNK = 256


def _make_sc_broadcast(batch: int, seq_len: int, d: int):
    info = plsc.get_sparse_core_info()
    nc, ns = info.num_cores, info.num_subcores
    rows_per_core = seq_len // nc
    r = _SP_CHUNK
    assert rows_per_core % r == 0
    nchunks = rows_per_core // r

    mesh = plsc.VectorSubcoreMesh(core_axis_name="c", subcore_axis_name="s")

    @functools.partial(
        pl.kernel,
        mesh=mesh,
        out_type=jax.ShapeDtypeStruct((batch, seq_len, d), jnp.float32),
        scratch_types=[
            pltpu.VMEM_SHARED((2, r, d), jnp.float32),
            pltpu.SemaphoreType.DMA,
            pltpu.SemaphoreType.DMA,
            pltpu.SemaphoreType.DMA,
            pltpu.SemaphoreType.DMA,
        ],
    )
    def k(pe_hbm, out_hbm, sbuf, g0, g1, s0, s1):
        cid = lax.axis_index("c")
        sid = lax.axis_index("s")

        @pl.when(sid == 0)
        def _():
            base = cid * rows_per_core
            gsem, ssem = [g0, g1], [s0, s1]

            def gather(c):
                return pltpu.make_async_copy(
                    pe_hbm.at[pl.ds(base + c * r, r)], sbuf.at[c % 2], gsem[c % 2])

            def scatters(c):
                return [
                    pltpu.make_async_copy(
                        sbuf.at[c % 2], out_hbm.at[b, pl.ds(base + c * r, r)],
                        ssem[c % 2])
                    for b in range(batch)
                ]

            pending = [None, None]
            gather(0).start()
            for c in range(nchunks):
                gather(c).wait()
                if c + 1 < nchunks:
                    if pending[(c + 1) % 2] is not None:
                        for cp in pending[(c + 1) % 2]:
                            cp.wait()
                        pending[(c + 1) % 2] = None
                    gather(c + 1).start()
                sc = scatters(c)
                for cp in sc:
                    cp.start()
                pending[c % 2] = sc
            for p in pending:
                if p is not None:
                    for cp in p:
                        cp.wait()

    return k


def kernel(x, pos_embed):
    batch, seq_len = x.shape[0], x.shape[1]
    d = pos_embed.shape[1]
    return _make_sc_broadcast(batch, seq_len, d)(pos_embed)


# speedup vs baseline: 1.5211x; 1.5211x over previous
import functools

import jax
import jax.numpy as jnp
from jax import lax
from jax.experimental import pallas as pl
from jax.experimental.pallas import tpu as pltpu
from jax.experimental.pallas import tpu_sc as plsc

_T_SIZES = (40, 40, 40, 32)
_SP_SIZES = (232, 232, 232, 232, 232, 232, 232, 192)


def _ring(gather, scatters, nchunks):
    pending = [None, None]
    gather(0).start()
    for c in range(nchunks):
        gather(c).wait()
        if c + 1 < nchunks:
            if pending[(c + 1) % 2] is not None:
                for cp in pending[(c + 1) % 2]:
                    cp.wait()
                pending[(c + 1) % 2] = None
            gather(c + 1).start()
        sc = scatters(c)
        for cp in sc:
            cp.start()
        pending[c % 2] = sc
    for p in pending:
        if p is not None:
            for cp in p:
                cp.wait()


def _make_sc_broadcast(batch: int, seq_len: int, d: int):
    info = plsc.get_sparse_core_info()
    nc, ns = info.num_cores, info.num_subcores
    assert seq_len % nc == 0
    rows_per_core = seq_len // nc

    t_offs = [sum(_T_SIZES[:i]) for i in range(len(_T_SIZES))]
    tile_rows = sum(_T_SIZES)
    t_rows_total = tile_rows * (ns - 1)
    sp_offs = [sum(_SP_SIZES[:i]) for i in range(len(_SP_SIZES))]
    sp_rows = sum(_SP_SIZES)
    assert t_rows_total + sp_rows == rows_per_core
    r_t = max(_T_SIZES)
    r_sp = max(_SP_SIZES)

    mesh = plsc.VectorSubcoreMesh(core_axis_name="c", subcore_axis_name="s")

    @functools.partial(
        pl.kernel,
        mesh=mesh,
        out_type=jax.ShapeDtypeStruct((batch, seq_len, d), jnp.float32),
        scratch_types=[
            pltpu.VMEM((r_t, d), jnp.float32),
            pltpu.VMEM((r_t, d), jnp.float32),
            pltpu.VMEM_SHARED((2, r_sp, d), jnp.float32),
            pltpu.SemaphoreType.DMA,
            pltpu.SemaphoreType.DMA,
            pltpu.SemaphoreType.DMA,
            pltpu.SemaphoreType.DMA,
        ],
    )
    def k(pe_hbm, out_hbm, buf0, buf1, sbuf, g0, g1, s0, s1):
        cid = lax.axis_index("c")
        sid = lax.axis_index("s")
        core_base = cid * rows_per_core
        gsem, ssem = [g0, g1], [s0, s1]

        @pl.when(sid > 0)
        def _tile_path():
            bufs = [buf0, buf1]
            base = core_base + (sid - 1) * tile_rows

            def gather(c):
                return pltpu.make_async_copy(
                    pe_hbm.at[pl.ds(base + t_offs[c], _T_SIZES[c])],
                    bufs[c % 2].at[pl.ds(0, _T_SIZES[c])], gsem[c % 2])

            def scatters(c):
                return [
                    pltpu.make_async_copy(
                        bufs[c % 2].at[pl.ds(0, _T_SIZES[c])],
                        out_hbm.at[b, pl.ds(base + t_offs[c], _T_SIZES[c])],
                        ssem[c % 2])
                    for b in range(batch)
                ]

            _ring(gather, scatters, len(_T_SIZES))

        @pl.when(sid == 0)
        def _spmem_path():
            base = core_base + t_rows_total

            def gather(c):
                return pltpu.make_async_copy(
                    pe_hbm.at[pl.ds(base + sp_offs[c], _SP_SIZES[c])],
                    sbuf.at[c % 2, pl.ds(0, _SP_SIZES[c])], gsem[c % 2])

            def scatters(c):
                return [
                    pltpu.make_async_copy(
                        sbuf.at[c % 2, pl.ds(0, _SP_SIZES[c])],
                        out_hbm.at[b, pl.ds(base + sp_offs[c], _SP_SIZES[c])],
                        ssem[c % 2])
                    for b in range(batch)
                ]

            _ring(gather, scatters, len(_SP_SIZES))

    return k


def kernel(x, pos_embed):
    batch, seq_len = x.shape[0], x.shape[1]
    d = pos_embed.shape[1]
    return _make_sc_broadcast(batch, seq_len, d)(pos_embed)
